# baseline (speedup 1.0000x reference)
import functools

import jax
import jax.numpy as jnp
from jax import lax
from jax.experimental import pallas as pl
from jax.experimental.pallas import tpu as pltpu
from jax.experimental.pallas import tpu_sc as plsc

D = 16
B = 16384
NPATHS = 5

NC = 2
NS = 16
L = 16
NW = NC * NS
BPW = B // NW
NGROUPS = BPW // L

_BITREV = (0, 8, 4, 12, 2, 10, 6, 14, 1, 9, 5, 13, 3, 11, 7, 15)


def _perm(x, idx):
    return x.at[idx].get(mode="promise_in_bounds")


def _sc_kernel(users_hbm, items_hbm, u0, v0, u1, v1, u2, v2, wu_flat,
               out_hbm,
               users_v, items_v, iw0, iw1, iw2, w0_v, w1_v, w2_v,
               ub0, vb0, ub1, vb1, ub2, vb2, out_v, sem):
    wid = lax.axis_index("s") * NC + lax.axis_index("c")
    base = wid * BPW

    pltpu.sync_copy(users_hbm.at[pl.ds(base, BPW)], users_v)
    pltpu.sync_copy(items_hbm.at[pl.ds(base, BPW)], items_v)

    copies = [
        pltpu.async_copy(u0.at[users_v], ub0, sem),
        pltpu.async_copy(v0.at[items_v], vb0, sem),
        pltpu.async_copy(u1.at[users_v], ub1, sem),
        pltpu.async_copy(v1.at[items_v], vb1, sem),
        pltpu.async_copy(u2.at[users_v], ub2, sem),
        pltpu.async_copy(v2.at[items_v], vb2, sem),
    ]

    def gidx(g, carry):
        p0 = g * L
        b5 = users_v[pl.ds(p0, L)] * NPATHS
        iw0[pl.ds(p0, L)] = b5
        iw1[pl.ds(p0, L)] = b5 + 1
        iw2[pl.ds(p0, L)] = b5 + 2
        return carry

    lax.fori_loop(0, NGROUPS, gidx, 0)
    copies.append(pltpu.async_copy(wu_flat.at[iw0], w0_v, sem))
    copies.append(pltpu.async_copy(wu_flat.at[iw1], w1_v, sem))
    copies.append(pltpu.async_copy(wu_flat.at[iw2], w2_v, sem))
    for c in copies:
        c.wait()

    lanes = lax.iota(jnp.int32, L)

    def group(g, carry):
        p0 = g * L
        num = jnp.zeros((L,), jnp.float32)
        den = jnp.zeros((L,), jnp.float32)
        for ub, vb, wv in ((ub0, vb0, w0_v), (ub1, vb1, w1_v),
                           (ub2, vb2, w2_v)):
            w = jnp.exp(wv[pl.ds(p0, L)])
            lvl = [ub[p0 + j, :] * vb[p0 + j, :] for j in _BITREV]
            for dist in (8, 4, 2, 1):
                m = (lanes & dist) == 0
                pidx = lanes ^ dist
                lvl = [jnp.where(m, a, _perm(b, pidx))
                       + jnp.where(m, _perm(a, pidx), b)
                       for a, b in zip(lvl[::2], lvl[1::2])]
            dots = lvl[0]
            num = num + w * dots
            den = den + w
        out_v[pl.ds(p0, L)] = num / den
        return carry

    lax.fori_loop(0, NGROUPS, group, 0)

    pltpu.sync_copy(out_v, out_hbm.at[pl.ds(base, BPW)])


@jax.jit
def _run(users, items, u0, v0, u1, v1, u2, v2, wu_flat):
    mesh = plsc.VectorSubcoreMesh(
        core_axis_name="c", subcore_axis_name="s",
        num_cores=NC, num_subcores=NS)
    f = functools.partial(
        pl.kernel,
        out_type=jax.ShapeDtypeStruct((B,), jnp.float32),
        mesh=mesh,
        compiler_params=pltpu.CompilerParams(use_tc_tiling_on_sc=False),
        scratch_types=[
            pltpu.VMEM((BPW,), jnp.int32),
            pltpu.VMEM((BPW,), jnp.int32),
            pltpu.VMEM((BPW,), jnp.int32),
            pltpu.VMEM((BPW,), jnp.int32),
            pltpu.VMEM((BPW,), jnp.int32),
            pltpu.VMEM((BPW,), jnp.float32),
            pltpu.VMEM((BPW,), jnp.float32),
            pltpu.VMEM((BPW,), jnp.float32),
            pltpu.VMEM((BPW, D), jnp.float32),
            pltpu.VMEM((BPW, D), jnp.float32),
            pltpu.VMEM((BPW, D), jnp.float32),
            pltpu.VMEM((BPW, D), jnp.float32),
            pltpu.VMEM((BPW, D), jnp.float32),
            pltpu.VMEM((BPW, D), jnp.float32),
            pltpu.VMEM((BPW,), jnp.float32),
            pltpu.SemaphoreType.DMA,
        ],
    )(_sc_kernel)
    return f(users, items, u0, v0, u1, v1, u2, v2, wu_flat)


def kernel(users, items, U_0, V_0, U_1, V_1, U_2, V_2, W_U):
    return _run(users.astype(jnp.int32), items.astype(jnp.int32),
                U_0, V_0, U_1, V_1, U_2, V_2, W_U.reshape(-1))


# speedup vs baseline: 1.1528x; 1.1528x over previous
import functools

import jax
import jax.numpy as jnp
from jax import lax
from jax.experimental import pallas as pl
from jax.experimental.pallas import tpu as pltpu
from jax.experimental.pallas import tpu_sc as plsc

D = 16
B = 16384
SRW = 128
RPS = SRW // D

NC = 2
NS = 16
L = 16
NW = NC * NS
BPW = B // NW
CHUNK = 128
NCHUNKS = BPW // CHUNK
GPC = CHUNK // L

_BITREV = (0, 8, 4, 12, 2, 10, 6, 14, 1, 9, 5, 13, 3, 11, 7, 15)


def _perm(x, idx):
    return x.at[idx].get(mode="promise_in_bounds")


def _sc_kernel(su_u_hbm, su_i_hbm, lo_u_hbm, lo_i_hbm, users_hbm,
               u0, v0, u1, v1, u2, v2, w0t, w1t, w2t,
               out_hbm,
               su_u_v, su_i_v, lo_u_v, lo_i_v, users_v,
               w0_v, w1_v, w2_v,
               ub0, vb0, ub1, vb1, ub2, vb2, out_v, sem, wsem):
    wid = lax.axis_index("s") * NC + lax.axis_index("c")
    base = wid * BPW

    pltpu.sync_copy(su_u_hbm.at[pl.ds(base, BPW)], su_u_v)
    pltpu.sync_copy(su_i_hbm.at[pl.ds(base, BPW)], su_i_v)
    pltpu.sync_copy(lo_u_hbm.at[pl.ds(base, BPW)], lo_u_v)
    pltpu.sync_copy(lo_i_hbm.at[pl.ds(base, BPW)], lo_i_v)
    pltpu.sync_copy(users_hbm.at[pl.ds(base, BPW)], users_v)

    wc0 = pltpu.async_copy(w0t.at[users_v], w0_v, wsem)
    wc1 = pltpu.async_copy(w1t.at[users_v], w1_v, wsem)
    wc2 = pltpu.async_copy(w2t.at[users_v], w2_v, wsem)

    lanes = lax.iota(jnp.int32, L)

    def chunk_body(c, carry):
        cb = c * CHUNK
        copies = [
            pltpu.async_copy(u0.at[su_u_v.at[pl.ds(cb, CHUNK)]], ub0, sem),
            pltpu.async_copy(v0.at[su_i_v.at[pl.ds(cb, CHUNK)]], vb0, sem),
            pltpu.async_copy(u1.at[su_u_v.at[pl.ds(cb, CHUNK)]], ub1, sem),
            pltpu.async_copy(v1.at[su_i_v.at[pl.ds(cb, CHUNK)]], vb1, sem),
            pltpu.async_copy(u2.at[su_u_v.at[pl.ds(cb, CHUNK)]], ub2, sem),
            pltpu.async_copy(v2.at[su_i_v.at[pl.ds(cb, CHUNK)]], vb2, sem),
        ]
        for cp in copies:
            cp.wait()

        def group(g, gcarry):
            p0 = cb + g * L
            num = jnp.zeros((L,), jnp.float32)
            den = jnp.zeros((L,), jnp.float32)
            lo_u16 = lo_u_v[pl.ds(p0, L)]
            lo_i16 = lo_i_v[pl.ds(p0, L)]
            urows = []
            virows = []
            for j in _BITREV:
                pj = g * L + j
                urows.append((pj, lo_u16[j]))
                virows.append((pj, lo_i16[j]))
            for ub, vb, wv in ((ub0, vb0, w0_v), (ub1, vb1, w1_v),
                               (ub2, vb2, w2_v)):
                w = jnp.exp(wv[pl.ds(p0, L)])
                lvl = [ub[pj, pl.ds(lo_u, D)] * vb[pj, pl.ds(lo_i, D)]
                       for (pj, lo_u), (_, lo_i) in zip(urows, virows)]
                for dist in (8, 4, 2, 1):
                    m = (lanes & dist) == 0
                    pidx = lanes ^ dist
                    lvl = [jnp.where(m, a, _perm(b, pidx))
                           + jnp.where(m, _perm(a, pidx), b)
                           for a, b in zip(lvl[::2], lvl[1::2])]
                dots = lvl[0]
                num = num + w * dots
                den = den + w
            out_v[pl.ds(p0, L)] = num / den
            return gcarry

        lax.fori_loop(0, GPC, group, 0)
        return carry

    wc0.wait()
    wc1.wait()
    wc2.wait()
    lax.fori_loop(0, NCHUNKS, chunk_body, 0)

    pltpu.sync_copy(out_v, out_hbm.at[pl.ds(base, BPW)])


@jax.jit
def _run(su_u, su_i, lo_u, lo_i, users,
         u0, v0, u1, v1, u2, v2, w0t, w1t, w2t):
    mesh = plsc.VectorSubcoreMesh(
        core_axis_name="c", subcore_axis_name="s",
        num_cores=NC, num_subcores=NS)
    f = functools.partial(
        pl.kernel,
        out_type=jax.ShapeDtypeStruct((B,), jnp.float32),
        mesh=mesh,
        compiler_params=pltpu.CompilerParams(use_tc_tiling_on_sc=False),
        scratch_types=[
            pltpu.VMEM((BPW,), jnp.int32),
            pltpu.VMEM((BPW,), jnp.int32),
            pltpu.VMEM((BPW,), jnp.int32),
            pltpu.VMEM((BPW,), jnp.int32),
            pltpu.VMEM((BPW,), jnp.int32),
            pltpu.VMEM((BPW,), jnp.float32),
            pltpu.VMEM((BPW,), jnp.float32),
            pltpu.VMEM((BPW,), jnp.float32),
            pltpu.VMEM((CHUNK, SRW), jnp.float32),
            pltpu.VMEM((CHUNK, SRW), jnp.float32),
            pltpu.VMEM((CHUNK, SRW), jnp.float32),
            pltpu.VMEM((CHUNK, SRW), jnp.float32),
            pltpu.VMEM((CHUNK, SRW), jnp.float32),
            pltpu.VMEM((CHUNK, SRW), jnp.float32),
            pltpu.VMEM((BPW,), jnp.float32),
            pltpu.SemaphoreType.DMA,
            pltpu.SemaphoreType.DMA,
        ],
    )(_sc_kernel)
    return f(su_u, su_i, lo_u, lo_i, users,
             u0, v0, u1, v1, u2, v2, w0t, w1t, w2t)


def kernel(users, items, U_0, V_0, U_1, V_1, U_2, V_2, W_U):
    users = users.astype(jnp.int32)
    items = items.astype(jnp.int32)
    su_u = users >> 3
    su_i = items >> 3
    lo_u = (users & 7) * D
    lo_i = (items & 7) * D
    n_super = U_0.shape[0] // RPS
    return _run(su_u, su_i, lo_u, lo_i, users,
                U_0.reshape(n_super, SRW), V_0.reshape(n_super, SRW),
                U_1.reshape(n_super, SRW), V_1.reshape(n_super, SRW),
                U_2.reshape(n_super, SRW), V_2.reshape(n_super, SRW),
                W_U[:, 0], W_U[:, 1], W_U[:, 2])


# speedup vs baseline: 1.1556x; 1.0024x over previous
import functools

import jax
import jax.numpy as jnp
from jax import lax
from jax.experimental import pallas as pl
from jax.experimental.pallas import tpu as pltpu
from jax.experimental.pallas import tpu_sc as plsc

D = 16
B = 16384
SRW = 128
RPS = SRW // D

NC = 2
NS = 16
L = 16
NW = NC * NS
BPW = B // NW
CHUNK = 128
NCHUNKS = BPW // CHUNK
GPC = CHUNK // L

_BITREV = (0, 8, 4, 12, 2, 10, 6, 14, 1, 9, 5, 13, 3, 11, 7, 15)


def _perm(x, idx):
    return x.at[idx].get(mode="promise_in_bounds")


def _sc_kernel(su_u_hbm, su_i_hbm, lo_u_hbm, lo_i_hbm, users_hbm,
               u0, v0, u1, v1, u2, v2, w0t, w1t, w2t,
               out_hbm,
               su_u_v, su_i_v, lo_u_v, lo_i_v, users_v,
               w0_v, w1_v, w2_v,
               ub0, vb0, ub1, vb1, ub2, vb2, out_v, sem, wsem):
    wid = lax.axis_index("s") * NC + lax.axis_index("c")
    base = wid * BPW

    pltpu.sync_copy(su_u_hbm.at[pl.ds(base, BPW)], su_u_v)
    pltpu.sync_copy(su_i_hbm.at[pl.ds(base, BPW)], su_i_v)
    pltpu.sync_copy(lo_u_hbm.at[pl.ds(base, BPW)], lo_u_v)
    pltpu.sync_copy(lo_i_hbm.at[pl.ds(base, BPW)], lo_i_v)
    pltpu.sync_copy(users_hbm.at[pl.ds(base, BPW)], users_v)

    wc0 = pltpu.async_copy(w0t.at[users_v], w0_v, wsem)
    wc1 = pltpu.async_copy(w1t.at[users_v], w1_v, wsem)
    wc2 = pltpu.async_copy(w2t.at[users_v], w2_v, wsem)

    lanes = lax.iota(jnp.int32, L)

    def chunk_body(c, carry):
        cb = c * CHUNK
        copies = [
            pltpu.async_copy(u0.at[su_u_v.at[pl.ds(cb, CHUNK)]], ub0, sem),
            pltpu.async_copy(v0.at[su_i_v.at[pl.ds(cb, CHUNK)]], vb0, sem),
            pltpu.async_copy(u1.at[su_u_v.at[pl.ds(cb, CHUNK)]], ub1, sem),
            pltpu.async_copy(v1.at[su_i_v.at[pl.ds(cb, CHUNK)]], vb1, sem),
            pltpu.async_copy(u2.at[su_u_v.at[pl.ds(cb, CHUNK)]], ub2, sem),
            pltpu.async_copy(v2.at[su_i_v.at[pl.ds(cb, CHUNK)]], vb2, sem),
        ]
        for cp in copies:
            cp.wait()

        def group(g, gcarry):
            p0 = cb + g * L
            num = jnp.zeros((L,), jnp.float32)
            den = jnp.zeros((L,), jnp.float32)
            lo_u16 = lo_u_v[pl.ds(p0, L)]
            lo_i16 = lo_i_v[pl.ds(p0, L)]
            urows = []
            virows = []
            for j in _BITREV:
                pj = g * L + j
                urows.append((pj, lo_u16[j]))
                virows.append((pj, lo_i16[j]))
            for ub, vb, wv in ((ub0, vb0, w0_v), (ub1, vb1, w1_v),
                               (ub2, vb2, w2_v)):
                w = jnp.exp(wv[pl.ds(p0, L)])
                lvl = [ub[pj, pl.ds(lo_u, D)] * vb[pj, pl.ds(lo_i, D)]
                       for (pj, lo_u), (_, lo_i) in zip(urows, virows)]
                for dist in (8, 4, 2, 1):
                    m = (lanes & dist) == 0
                    pidx = lanes ^ dist
                    lvl = [jnp.where(m, a, _perm(b, pidx))
                           + jnp.where(m, _perm(a, pidx), b)
                           for a, b in zip(lvl[::2], lvl[1::2])]
                dots = lvl[0]
                num = num + w * dots
                den = den + w
            out_v[pl.ds(p0, L)] = num / den
            return gcarry

        lax.fori_loop(0, GPC, group, 0)
        return carry

    wc0.wait()
    wc1.wait()
    wc2.wait()
    lax.fori_loop(0, NCHUNKS, chunk_body, 0)

    pltpu.sync_copy(out_v, out_hbm.at[pl.ds(base, BPW)])


@jax.jit
def _run(su_u, su_i, lo_u, lo_i, users,
         u0, v0, u1, v1, u2, v2, w0t, w1t, w2t):
    mesh = plsc.VectorSubcoreMesh(
        core_axis_name="c", subcore_axis_name="s",
        num_cores=NC, num_subcores=NS)
    f = functools.partial(
        pl.kernel,
        out_type=jax.ShapeDtypeStruct((B,), jnp.float32),
        mesh=mesh,
        compiler_params=pltpu.CompilerParams(use_tc_tiling_on_sc=True),
        scratch_types=[
            pltpu.VMEM((BPW,), jnp.int32),
            pltpu.VMEM((BPW,), jnp.int32),
            pltpu.VMEM((BPW,), jnp.int32),
            pltpu.VMEM((BPW,), jnp.int32),
            pltpu.VMEM((BPW,), jnp.int32),
            pltpu.VMEM((BPW,), jnp.float32),
            pltpu.VMEM((BPW,), jnp.float32),
            pltpu.VMEM((BPW,), jnp.float32),
            pltpu.VMEM((CHUNK, SRW), jnp.float32),
            pltpu.VMEM((CHUNK, SRW), jnp.float32),
            pltpu.VMEM((CHUNK, SRW), jnp.float32),
            pltpu.VMEM((CHUNK, SRW), jnp.float32),
            pltpu.VMEM((CHUNK, SRW), jnp.float32),
            pltpu.VMEM((CHUNK, SRW), jnp.float32),
            pltpu.VMEM((BPW,), jnp.float32),
            pltpu.SemaphoreType.DMA,
            pltpu.SemaphoreType.DMA,
        ],
    )(_sc_kernel)
    return f(su_u, su_i, lo_u, lo_i, users,
             u0, v0, u1, v1, u2, v2, w0t, w1t, w2t)


def kernel(users, items, U_0, V_0, U_1, V_1, U_2, V_2, W_U):
    users = users.astype(jnp.int32)
    items = items.astype(jnp.int32)
    su_u = users >> 3
    su_i = items >> 3
    lo_u = (users & 7) * D
    lo_i = (items & 7) * D
    n_super = U_0.shape[0] // RPS
    return _run(su_u, su_i, lo_u, lo_i, users,
                U_0.reshape(n_super, SRW), V_0.reshape(n_super, SRW),
                U_1.reshape(n_super, SRW), V_1.reshape(n_super, SRW),
                U_2.reshape(n_super, SRW), V_2.reshape(n_super, SRW),
                W_U[:, 0], W_U[:, 1], W_U[:, 2])
